# baseline (speedup 1.0000x reference)
import jax
import jax.numpy as jnp
from jax import lax
from jax.experimental import pallas as pl
from jax.experimental.pallas import tpu as pltpu
from jax.experimental.pallas import tpu_sc as plsc

N_ROWS = 320000
N_COLS = 128
N_GROUPS = 10000
HALF_G = N_GROUPS // 2
ACC_H = 5056
CHUNK = 128
N_CHUNKS = N_ROWS // CHUNK
NC, NS = 2, 16
STRIPE = 320
L = 16
SEARCH_STEPS = 12


def _sc_body(x_hbm, grp_hbm, zx_hbm, ones_hbm, psum_hbm, pcnt_hbm,
             xb0, xb1, onesbuf, ib0, ib1, i2b0, i2b1, acc, cnt,
             semx0, semx1, sems0, semc0, sems1, semc1):
    cid = lax.axis_index("c")
    sid = lax.axis_index("s")
    lo = cid * HALF_G
    hi = lo + HALF_G

    pltpu.sync_copy(zx_hbm, xb0)
    base = jnp.minimum(sid * STRIPE, ACC_H - STRIPE)
    for off, n in ((0, 128), (128, 128), (256, 64)):
        r0 = base + off
        pltpu.sync_copy(xb0.at[pl.ds(0, n)], acc.at[pl.ds(r0, n)])
        pltpu.sync_copy(xb0.at[pl.ds(0, n)], cnt.at[pl.ds(r0, n)])
    pltpu.sync_copy(ones_hbm, onesbuf)

    def sbody(_, carry):
        s_lo, s_hi = carry
        mid = lax.div(s_lo + s_hi, 2)
        pltpu.sync_copy(grp_hbm.at[pl.ds(mid * CHUNK, L)], ib0.at[pl.ds(0, L)])
        v = ib0[pl.ds(0, L)][0]
        p = v >= HALF_G
        return jnp.where(p, s_lo, mid + 1), jnp.where(p, mid, s_hi)

    _, cb = lax.fori_loop(0, SEARCH_STEPS, sbody, (0, N_CHUNKS))
    start = jnp.where(cid == 0, 0, jnp.maximum(cb - 1, 0))
    end = jnp.where(cid == 0, cb, N_CHUNKS)
    start_c = start + sid
    n = jnp.maximum(0, lax.div(end - start_c + (NS - 1), NS))

    plsc.subcore_barrier()

    def cofs(i):
        return (start_c + NS * i) * CHUNK

    def cofs_clamped(i):
        return (start_c + NS * jnp.minimum(i, n - 1)) * CHUNK

    def remap(ib, i2b):
        for j in range(CHUNK // L):
            v = ib[pl.ds(j * L, L)]
            in_range = jnp.logical_and(v >= lo, v < hi)
            i2b[pl.ds(j * L, L)] = jnp.where(in_range, v - lo, HALF_G)

    def scatter(xb, i2b, ss, sc_):
        hs = pltpu.async_copy(xb, acc.at[i2b], ss, add=True)
        pltpu.async_copy(onesbuf, cnt.at[i2b], sc_, add=True)
        return hs

    def wait_gather(xb, sem):
        pltpu.make_async_copy(x_hbm.at[pl.ds(0, CHUNK)], xb, sem).wait()

    def wait_ones(i2b, sem):
        pltpu.make_async_copy(onesbuf, cnt.at[i2b], sem).wait()

    @pl.when(n > 0)
    def _():
        r0 = cofs(0)
        r1 = cofs_clamped(1)
        pltpu.sync_copy(grp_hbm.at[pl.ds(r0, CHUNK)], ib0)
        pltpu.sync_copy(grp_hbm.at[pl.ds(r1, CHUNK)], ib1)
        pltpu.async_copy(x_hbm.at[pl.ds(r0, CHUNK)], xb0, semx0)
        pltpu.async_copy(x_hbm.at[pl.ds(r1, CHUNK)], xb1, semx1)

        def body(k, carry):
            @pl.when(k > 0)
            def _():
                wait_ones(i2b0, semc0)
                wait_ones(i2b1, semc1)

            remap(ib0, i2b0)
            remap(ib1, i2b1)
            wait_gather(xb0, semx0)
            h0 = scatter(xb0, i2b0, sems0, semc0)
            wait_gather(xb1, semx1)
            h1 = scatter(xb1, i2b1, sems1, semc1)
            ra = cofs_clamped(2 * k + 2)
            rb = cofs_clamped(2 * k + 3)
            pltpu.sync_copy(grp_hbm.at[pl.ds(ra, CHUNK)], ib0)
            pltpu.sync_copy(grp_hbm.at[pl.ds(rb, CHUNK)], ib1)
            h0.wait()
            pltpu.async_copy(x_hbm.at[pl.ds(ra, CHUNK)], xb0, semx0)
            h1.wait()
            pltpu.async_copy(x_hbm.at[pl.ds(rb, CHUNK)], xb1, semx1)
            return carry

        nt = lax.div(n, 2)
        lax.fori_loop(0, nt, body, 0)

        wait_gather(xb0, semx0)
        wait_gather(xb1, semx1)

        @pl.when(nt > 0)
        def _():
            wait_ones(i2b0, semc0)
            wait_ones(i2b1, semc1)

        @pl.when(lax.rem(n, 2) == 1)
        def _():
            remap(ib0, i2b0)
            h = scatter(xb0, i2b0, sems0, semc0)
            h.wait()
            wait_ones(i2b0, semc0)


    plsc.subcore_barrier()

    for off, n2 in ((0, 128), (128, 128), (256, 64)):
        r0 = jnp.minimum(sid * STRIPE, ACC_H - STRIPE) + off
        pltpu.sync_copy(acc.at[pl.ds(r0, n2)], xb0.at[pl.ds(0, n2)])
        pltpu.sync_copy(xb0.at[pl.ds(0, n2)],
                        psum_hbm.at[pl.ds(cid * ACC_H + r0, n2)])
        pltpu.sync_copy(cnt.at[pl.ds(r0, n2)], xb1.at[pl.ds(0, n2)])
        pltpu.sync_copy(xb1.at[pl.ds(0, n2)],
                        pcnt_hbm.at[pl.ds(cid * ACC_H + r0, n2)])


def _merge_body(ps_ref, pc_ref, out_ref):
    out_ref[...] = ps_ref[0] / pc_ref[0]


def kernel(x, group):
    grp = group.astype(jnp.int32)
    zx = jnp.zeros((CHUNK, N_COLS), jnp.float32)
    ones = jnp.ones((CHUNK, N_COLS), jnp.float32)

    sc = pl.kernel(
        _sc_body,
        out_type=(
            jax.ShapeDtypeStruct((NC * ACC_H, N_COLS), jnp.float32),
            jax.ShapeDtypeStruct((NC * ACC_H, N_COLS), jnp.float32),
        ),
        mesh=plsc.VectorSubcoreMesh(core_axis_name="c", subcore_axis_name="s"),
        scratch_types=[
            pltpu.VMEM((CHUNK, N_COLS), jnp.float32),
            pltpu.VMEM((CHUNK, N_COLS), jnp.float32),
            pltpu.VMEM((CHUNK, N_COLS), jnp.float32),
            pltpu.VMEM((CHUNK,), jnp.int32),
            pltpu.VMEM((CHUNK,), jnp.int32),
            pltpu.VMEM((CHUNK,), jnp.int32),
            pltpu.VMEM((CHUNK,), jnp.int32),
            pltpu.VMEM_SHARED((ACC_H, N_COLS), jnp.float32),
            pltpu.VMEM_SHARED((ACC_H, N_COLS), jnp.float32),
            pltpu.SemaphoreType.DMA,
            pltpu.SemaphoreType.DMA,
            pltpu.SemaphoreType.DMA,
            pltpu.SemaphoreType.DMA,
            pltpu.SemaphoreType.DMA,
            pltpu.SemaphoreType.DMA,
        ],
    )
    psum, pcnt = sc(x, grp, zx, ones)
    psum = psum.reshape(NC, ACC_H, N_COLS)
    pcnt = pcnt.reshape(NC, ACC_H, N_COLS)

    nblk = 10
    blk = N_GROUPS // nblk
    out = pl.pallas_call(
        _merge_body,
        grid=(nblk,),
        in_specs=[
            pl.BlockSpec((1, blk, N_COLS), lambda i: (i // 5, i % 5, 0)),
            pl.BlockSpec((1, blk, N_COLS), lambda i: (i // 5, i % 5, 0)),
        ],
        out_specs=pl.BlockSpec((blk, N_COLS), lambda i: (i, 0)),
        out_shape=jax.ShapeDtypeStruct((N_GROUPS, N_COLS), jnp.float32),
    )(psum, pcnt)
    return out


# speedup vs baseline: 1.0047x; 1.0047x over previous
import jax
import jax.numpy as jnp
from jax import lax
from jax.experimental import pallas as pl
from jax.experimental.pallas import tpu as pltpu
from jax.experimental.pallas import tpu_sc as plsc

N_ROWS = 320000
N_COLS = 128
N_GROUPS = 10000
HALF_G = N_GROUPS // 2
ACC_H = 5056
CHUNK = 128
N_CHUNKS = N_ROWS // CHUNK
NC, NS = 2, 16
STRIPE = 320
L = 16
SEARCH_STEPS = 12


def _sc_body(x_hbm, grp_hbm, zx_hbm, ones_hbm, psum_hbm, pcnt_hbm,
             xb0, xb1, onesbuf, ib0, ib1, i2b0, i2b1, acc, cnt,
             semx0, semx1, sems0, semc0, sems1, semc1):
    cid = lax.axis_index("c")
    sid = lax.axis_index("s")
    lo = cid * HALF_G
    hi = lo + HALF_G

    pltpu.sync_copy(zx_hbm, xb0)
    base = jnp.minimum(sid * STRIPE, ACC_H - STRIPE)
    for off, n in ((0, 128), (128, 128), (256, 64)):
        r0 = base + off
        pltpu.sync_copy(xb0.at[pl.ds(0, n)], acc.at[pl.ds(r0, n)])
        pltpu.sync_copy(xb0.at[pl.ds(0, n)], cnt.at[pl.ds(r0, n)])
    pltpu.sync_copy(ones_hbm, onesbuf)

    def sbody(_, carry):
        s_lo, s_hi = carry
        mid = lax.div(s_lo + s_hi, 2)
        pltpu.sync_copy(grp_hbm.at[pl.ds(mid * CHUNK, L)], ib0.at[pl.ds(0, L)])
        v = ib0[pl.ds(0, L)][0]
        p = v >= HALF_G
        return jnp.where(p, s_lo, mid + 1), jnp.where(p, mid, s_hi)

    _, cb = lax.fori_loop(0, SEARCH_STEPS, sbody, (0, N_CHUNKS))
    start = jnp.where(cid == 0, 0, jnp.maximum(cb - 1, 0))
    end = jnp.where(cid == 0, cb, N_CHUNKS)
    start_c = start + sid
    n = jnp.maximum(0, lax.div(end - start_c + (NS - 1), NS))

    plsc.subcore_barrier()

    def cofs(i):
        return (start_c + NS * i) * CHUNK

    def cofs_clamped(i):
        return (start_c + NS * jnp.minimum(i, n - 1)) * CHUNK

    def remap(ib, i2b):
        for j in range(CHUNK // L):
            v = ib[pl.ds(j * L, L)]
            in_range = jnp.logical_and(v >= lo, v < hi)
            i2b[pl.ds(j * L, L)] = jnp.where(in_range, v - lo, HALF_G)

    def scatter(xb, i2b, ss, sc_):
        hs = pltpu.async_copy(xb, acc.at[i2b], ss, add=True)
        pltpu.async_copy(onesbuf, cnt.at[i2b], sc_, add=True)
        return hs

    def wait_gather(xb, sem):
        pltpu.make_async_copy(x_hbm.at[pl.ds(0, CHUNK)], xb, sem).wait()

    def wait_ones(i2b, sem):
        pltpu.make_async_copy(onesbuf, cnt.at[i2b], sem).wait()

    @pl.when(n > 0)
    def _():
        r0 = cofs(0)
        r1 = cofs_clamped(1)
        pltpu.sync_copy(grp_hbm.at[pl.ds(r0, CHUNK)], ib0)
        pltpu.sync_copy(grp_hbm.at[pl.ds(r1, CHUNK)], ib1)
        pltpu.async_copy(x_hbm.at[pl.ds(r0, CHUNK)], xb0, semx0)
        pltpu.async_copy(x_hbm.at[pl.ds(r1, CHUNK)], xb1, semx1)

        def body(k, carry):
            @pl.when(k > 0)
            def _():
                wait_ones(i2b0, semc0)
                wait_ones(i2b1, semc1)

            remap(ib0, i2b0)
            remap(ib1, i2b1)
            wait_gather(xb0, semx0)
            h0 = scatter(xb0, i2b0, sems0, semc0)
            wait_gather(xb1, semx1)
            h1 = scatter(xb1, i2b1, sems1, semc1)
            ra = cofs_clamped(2 * k + 2)
            rb = cofs_clamped(2 * k + 3)
            pltpu.sync_copy(grp_hbm.at[pl.ds(ra, CHUNK)], ib0)
            pltpu.sync_copy(grp_hbm.at[pl.ds(rb, CHUNK)], ib1)
            h0.wait()
            pltpu.async_copy(x_hbm.at[pl.ds(ra, CHUNK)], xb0, semx0)
            h1.wait()
            pltpu.async_copy(x_hbm.at[pl.ds(rb, CHUNK)], xb1, semx1)
            return carry

        nt = lax.div(n, 2)
        lax.fori_loop(0, nt, body, 0)

        wait_gather(xb0, semx0)
        wait_gather(xb1, semx1)

        @pl.when(nt > 0)
        def _():
            wait_ones(i2b0, semc0)
            wait_ones(i2b1, semc1)

        @pl.when(lax.rem(n, 2) == 1)
        def _():
            remap(ib0, i2b0)
            h = scatter(xb0, i2b0, sems0, semc0)
            h.wait()
            wait_ones(i2b0, semc0)

    plsc.subcore_barrier()

    r0 = jnp.minimum(sid * STRIPE, ACC_H - STRIPE)
    pltpu.sync_copy(acc.at[pl.ds(r0, STRIPE)],
                    psum_hbm.at[pl.ds(cid * ACC_H + r0, STRIPE)])
    pltpu.sync_copy(cnt.at[pl.ds(r0, STRIPE)],
                    pcnt_hbm.at[pl.ds(cid * ACC_H + r0, STRIPE)])


def _merge_body(ps_ref, pc_ref, out_ref):
    out_ref[...] = ps_ref[0] / pc_ref[0]


def kernel(x, group):
    grp = group.astype(jnp.int32)
    zx = jnp.zeros((CHUNK, N_COLS), jnp.float32)
    ones = jnp.ones((CHUNK, N_COLS), jnp.float32)

    sc = pl.kernel(
        _sc_body,
        out_type=(
            jax.ShapeDtypeStruct((NC * ACC_H, N_COLS), jnp.float32),
            jax.ShapeDtypeStruct((NC * ACC_H, N_COLS), jnp.float32),
        ),
        mesh=plsc.VectorSubcoreMesh(core_axis_name="c", subcore_axis_name="s"),
        scratch_types=[
            pltpu.VMEM((CHUNK, N_COLS), jnp.float32),
            pltpu.VMEM((CHUNK, N_COLS), jnp.float32),
            pltpu.VMEM((CHUNK, N_COLS), jnp.float32),
            pltpu.VMEM((CHUNK,), jnp.int32),
            pltpu.VMEM((CHUNK,), jnp.int32),
            pltpu.VMEM((CHUNK,), jnp.int32),
            pltpu.VMEM((CHUNK,), jnp.int32),
            pltpu.VMEM_SHARED((ACC_H, N_COLS), jnp.float32),
            pltpu.VMEM_SHARED((ACC_H, N_COLS), jnp.float32),
            pltpu.SemaphoreType.DMA,
            pltpu.SemaphoreType.DMA,
            pltpu.SemaphoreType.DMA,
            pltpu.SemaphoreType.DMA,
            pltpu.SemaphoreType.DMA,
            pltpu.SemaphoreType.DMA,
        ],
    )
    psum, pcnt = sc(x, grp, zx, ones)
    psum = psum.reshape(NC, ACC_H, N_COLS)
    pcnt = pcnt.reshape(NC, ACC_H, N_COLS)

    nblk = 10
    blk = N_GROUPS // nblk
    out = pl.pallas_call(
        _merge_body,
        grid=(nblk,),
        in_specs=[
            pl.BlockSpec((1, blk, N_COLS), lambda i: (i // 5, i % 5, 0)),
            pl.BlockSpec((1, blk, N_COLS), lambda i: (i // 5, i % 5, 0)),
        ],
        out_specs=pl.BlockSpec((blk, N_COLS), lambda i: (i, 0)),
        out_shape=jax.ShapeDtypeStruct((N_GROUPS, N_COLS), jnp.float32),
    )(psum, pcnt)
    return out
